# Initial kernel scaffold; baseline (speedup 1.0000x reference)
import jax
import jax.numpy as jnp
from jax.experimental import pallas as pl


def kernel(x, edge_attr, W_i_w, W_i_b, W_h_w, W_h_b, W_o_w, W_o_b, edge_index, rev_edge_index):
    raise NotImplementedError("write your pallas kernel here")


# speedup vs baseline: 1.9915x; 1.9915x over previous
import functools

import jax
import jax.numpy as jnp
from jax import lax
from jax.experimental import pallas as pl
from jax.experimental.pallas import tpu as pltpu
from jax.experimental.pallas import tpu_sc as plsc

F32 = jnp.float32


def _mm_body(x_ref, w_ref, o_ref):
    o_ref[...] = jnp.dot(x_ref[...], w_ref[...], preferred_element_type=F32)


def _tc_matmul(x, w, bn):
    n, d = x.shape
    h = w.shape[1]
    return pl.pallas_call(
        _mm_body,
        grid=(n // bn,),
        in_specs=[
            pl.BlockSpec((bn, d), lambda i: (i, 0)),
            pl.BlockSpec((d, h), lambda i: (0, 0)),
        ],
        out_specs=pl.BlockSpec((bn, h), lambda i: (i, 0)),
        out_shape=jax.ShapeDtypeStruct((n, h), F32),
    )(x, w)


def _h0k1_body(ea_ref, pg_ref, wet_ref, bi_ref, wht_ref, h0_ref, k1_ref):
    h0 = (
        jnp.dot(ea_ref[...], wet_ref[...], preferred_element_type=F32)
        + bi_ref[...]
        + pg_ref[...]
    )
    h0_ref[...] = h0
    k1_ref[...] = jnp.dot(
        jnp.maximum(h0, 0.0), wht_ref[...], preferred_element_type=F32
    )


def _tc_h0k1(ea, pg, wet, bi2, wht, be):
    e, de = ea.shape
    hid = wet.shape[1]
    out_sd = jax.ShapeDtypeStruct((e, hid), F32)
    return pl.pallas_call(
        _h0k1_body,
        grid=(e // be,),
        in_specs=[
            pl.BlockSpec((be, de), lambda i: (i, 0)),
            pl.BlockSpec((be, hid), lambda i: (i, 0)),
            pl.BlockSpec((de, hid), lambda i: (0, 0)),
            pl.BlockSpec((1, hid), lambda i: (0, 0)),
            pl.BlockSpec((hid, hid), lambda i: (0, 0)),
        ],
        out_specs=[
            pl.BlockSpec((be, hid), lambda i: (i, 0)),
            pl.BlockSpec((be, hid), lambda i: (i, 0)),
        ],
        out_shape=[out_sd, out_sd],
    )(ea, pg, wet, bi2, wht)


def _upd_mm_body(h0_ref, sg_ref, kr_ref, bh_ref, wht_ref, k_ref):
    h = jnp.maximum(h0_ref[...] + sg_ref[...] - kr_ref[...] + bh_ref[...], 0.0)
    k_ref[...] = jnp.dot(h, wht_ref[...], preferred_element_type=F32)


def _tc_update_matmul(h0, sg, kr, bh2, wht, be):
    e, hid = h0.shape
    return pl.pallas_call(
        _upd_mm_body,
        grid=(e // be,),
        in_specs=[
            pl.BlockSpec((be, hid), lambda i: (i, 0)),
            pl.BlockSpec((be, hid), lambda i: (i, 0)),
            pl.BlockSpec((be, hid), lambda i: (i, 0)),
            pl.BlockSpec((1, hid), lambda i: (0, 0)),
            pl.BlockSpec((hid, hid), lambda i: (0, 0)),
        ],
        out_specs=pl.BlockSpec((be, hid), lambda i: (i, 0)),
        out_shape=jax.ShapeDtypeStruct((e, hid), F32),
    )(h0, sg, kr, bh2, wht)


def _upd_body(h0_ref, sg_ref, kr_ref, bh_ref, h_ref):
    h_ref[...] = jnp.maximum(h0_ref[...] + sg_ref[...] - kr_ref[...] + bh_ref[...], 0.0)


def _tc_update(h0, sg, kr, bh2, be):
    e, hid = h0.shape
    return pl.pallas_call(
        _upd_body,
        grid=(e // be,),
        in_specs=[
            pl.BlockSpec((be, hid), lambda i: (i, 0)),
            pl.BlockSpec((be, hid), lambda i: (i, 0)),
            pl.BlockSpec((be, hid), lambda i: (i, 0)),
            pl.BlockSpec((1, hid), lambda i: (0, 0)),
        ],
        out_specs=pl.BlockSpec((be, hid), lambda i: (i, 0)),
        out_shape=jax.ShapeDtypeStruct((e, hid), F32),
    )(h0, sg, kr, bh2)


def _merge_body(p_ref, o_ref):
    o_ref[...] = p_ref[0] + p_ref[1]


def _tc_merge(parts, bn):
    nc, n, hid = parts.shape
    return pl.pallas_call(
        _merge_body,
        grid=(n // bn,),
        in_specs=[pl.BlockSpec((nc, bn, hid), lambda i: (0, i, 0))],
        out_specs=pl.BlockSpec((bn, hid), lambda i: (i, 0)),
        out_shape=jax.ShapeDtypeStruct((n, hid), F32),
    )(parts)


def _final_body(x_ref, sfp_ref, woxt_ref, womt_ref, bo_ref, o_ref):
    sf = sfp_ref[0] + sfp_ref[1]
    rs = jnp.sum(sf, axis=1, keepdims=True)
    m = jnp.where(rs == 0.0, x_ref[...], sf)
    o_ref[...] = jnp.maximum(
        jnp.dot(x_ref[...], woxt_ref[...], preferred_element_type=F32)
        + jnp.dot(m, womt_ref[...], preferred_element_type=F32)
        + bo_ref[...],
        0.0,
    )


def _tc_final(x, sfp, woxt, womt, bo2, bn):
    n, d = x.shape
    hid = womt.shape[1]
    return pl.pallas_call(
        _final_body,
        grid=(n // bn,),
        in_specs=[
            pl.BlockSpec((bn, d), lambda i: (i, 0)),
            pl.BlockSpec((2, bn, hid), lambda i: (0, i, 0)),
            pl.BlockSpec((d, hid), lambda i: (0, 0)),
            pl.BlockSpec((hid, hid), lambda i: (0, 0)),
            pl.BlockSpec((1, hid), lambda i: (0, 0)),
        ],
        out_specs=pl.BlockSpec((bn, hid), lambda i: (i, 0)),
        out_shape=jax.ShapeDtypeStruct((n, hid), F32),
    )(x, sfp, woxt, womt, bo2)


def _sc_gather1(table, idx3, e_total, nc):
    nw, chunks, c = idx3.shape
    perw = chunks * c
    hid = table.shape[1]
    mesh = plsc.VectorSubcoreMesh(core_axis_name="c", subcore_axis_name="s")

    @functools.partial(
        pl.kernel,
        mesh=mesh,
        out_type=jax.ShapeDtypeStruct((e_total, hid), F32),
        scratch_types=[
            pltpu.VMEM((chunks, c), jnp.int32),
            pltpu.VMEM((c, hid), F32),
            pltpu.SemaphoreType.DMA,
        ],
    )
    def k(table_h, idx_h, out_h, idx_v, rows_v, sem):
        cid = lax.axis_index("c")
        sid = lax.axis_index("s")
        wid = sid * nc + cid
        pltpu.sync_copy(idx_h.at[wid], idx_v)

        def body(ci, carry):
            base = wid * perw + ci * c
            pltpu.async_copy(table_h.at[idx_v.at[ci]], rows_v, sem).wait()
            pltpu.sync_copy(rows_v, out_h.at[pl.ds(base, c)])
            return carry

        lax.fori_loop(0, chunks, body, 0)

    return k(table, idx3)


def _sc_gather2(table_a, idx_a3, table_b, idx_b3, e_total, nc):
    nw, chunks, c = idx_a3.shape
    perw = chunks * c
    hid = table_a.shape[1]
    mesh = plsc.VectorSubcoreMesh(core_axis_name="c", subcore_axis_name="s")
    out_sd = jax.ShapeDtypeStruct((e_total, hid), F32)

    @functools.partial(
        pl.kernel,
        mesh=mesh,
        out_type=[out_sd, out_sd],
        scratch_types=[
            pltpu.VMEM((chunks, c), jnp.int32),
            pltpu.VMEM((chunks, c), jnp.int32),
            pltpu.VMEM((c, hid), F32),
            pltpu.VMEM((c, hid), F32),
            pltpu.SemaphoreType.DMA,
            pltpu.SemaphoreType.DMA,
        ],
    )
    def k(ta_h, ia_h, tb_h, ib_h, oa_h, ob_h, ia_v, ib_v, a_v, b_v, sem_a, sem_b):
        cid = lax.axis_index("c")
        sid = lax.axis_index("s")
        wid = sid * nc + cid
        pltpu.sync_copy(ia_h.at[wid], ia_v)
        pltpu.sync_copy(ib_h.at[wid], ib_v)

        def body(ci, carry):
            base = wid * perw + ci * c
            cp_a = pltpu.async_copy(ta_h.at[ia_v.at[ci]], a_v, sem_a)
            cp_b = pltpu.async_copy(tb_h.at[ib_v.at[ci]], b_v, sem_b)
            cp_a.wait()
            cp_b.wait()
            pltpu.sync_copy(a_v, oa_h.at[pl.ds(base, c)])
            pltpu.sync_copy(b_v, ob_h.at[pl.ds(base, c)])
            return carry

        lax.fori_loop(0, chunks, body, 0)

    return k(table_a, idx_a3, table_b, idx_b3)


def _sc_scatter_add(rows, dst3, n_pad, nc, ns):
    e_total, hid = rows.shape
    nw, chunks, c = dst3.shape
    perw = chunks * c
    stripe = n_pad // ns
    mesh = plsc.VectorSubcoreMesh(core_axis_name="c", subcore_axis_name="s")

    n_full, rem = stripe // c, stripe % c

    @functools.partial(
        pl.kernel,
        mesh=mesh,
        out_type=jax.ShapeDtypeStruct((nc, n_pad, hid), F32),
        scratch_types=[
            pltpu.VMEM((chunks, c), jnp.int32),
            pltpu.VMEM((c, hid), F32),
            pltpu.VMEM_SHARED((n_pad, hid), F32),
            pltpu.SemaphoreType.DMA,
        ],
    )
    def k(rows_h, dst_h, out_h, idx_v, rows_v, acc_sh, sem):
        cid = lax.axis_index("c")
        sid = lax.axis_index("s")
        wid = sid * nc + cid
        z16 = jnp.zeros((16,), F32)

        def zrow(i, carry):
            for j in range(hid // 16):
                rows_v[i, pl.ds(j * 16, 16)] = z16
            return carry

        lax.fori_loop(0, c, zrow, 0)
        for t in range(n_full):
            pltpu.sync_copy(rows_v, acc_sh.at[pl.ds(sid * stripe + t * c, c)])
        if rem:
            pltpu.sync_copy(
                rows_v.at[pl.ds(0, rem)],
                acc_sh.at[pl.ds(sid * stripe + n_full * c, rem)],
            )
        plsc.subcore_barrier()

        pltpu.sync_copy(dst_h.at[wid], idx_v)

        def body(ci, carry):
            base = wid * perw + ci * c
            pltpu.sync_copy(rows_h.at[pl.ds(base, c)], rows_v)
            pltpu.sync_copy(rows_v, acc_sh.at[idx_v.at[ci]], add=True)
            return carry

        lax.fori_loop(0, chunks, body, 0)
        plsc.subcore_barrier()

        for t in range(n_full):
            pltpu.sync_copy(acc_sh.at[pl.ds(sid * stripe + t * c, c)], rows_v)
            pltpu.sync_copy(rows_v, out_h.at[cid, pl.ds(sid * stripe + t * c, c)])
        if rem:
            pltpu.sync_copy(
                acc_sh.at[pl.ds(sid * stripe + n_full * c, rem)],
                rows_v.at[pl.ds(0, rem)],
            )
            pltpu.sync_copy(
                rows_v.at[pl.ds(0, rem)],
                out_h.at[cid, pl.ds(sid * stripe + n_full * c, rem)],
            )

    return k(rows, dst3)


def kernel(x, edge_attr, W_i_w, W_i_b, W_h_w, W_h_b, W_o_w, W_o_b, edge_index, rev_edge_index):
    n, d = x.shape
    e, de = edge_attr.shape
    hid = W_i_w.shape[0]

    info = plsc.get_sparse_core_info()
    nc, ns = info.num_cores, info.num_subcores
    nw = nc * ns
    perw = e // nw
    c = 80
    chunks = perw // c

    src = edge_index[0].astype(jnp.int32)
    dst = edge_index[1].astype(jnp.int32)
    rev = rev_edge_index.astype(jnp.int32)
    src3 = src.reshape(nw, chunks, c)
    dst3 = dst.reshape(nw, chunks, c)
    rev3 = rev.reshape(nw, chunks, c)

    wxt = W_i_w[:, :d].T
    wet = W_i_w[:, d:].T
    wht = W_h_w.T
    woxt = W_o_w[:, :d].T
    womt = W_o_w[:, d:].T
    bi2 = W_i_b.reshape(1, hid)
    bh2 = W_h_b.reshape(1, hid)
    bo2 = W_o_b.reshape(1, hid)

    bn = 1000
    be = 1000
    stripe = (-(-n // ns) + 7) // 8 * 8
    n_pad = stripe * ns
    bn_pad = 2 * stripe

    p = _tc_matmul(x, wxt, bn)
    pg = _sc_gather1(p, src3, e, nc)
    h0, k1 = _tc_h0k1(edge_attr, pg, wet, bi2, wht, be)
    s1p = _sc_scatter_add(k1, dst3, n_pad, nc, ns)
    s1 = _tc_merge(s1p, bn_pad)
    sg1, kr1 = _sc_gather2(s1, src3, k1, rev3, e, nc)
    k2 = _tc_update_matmul(h0, sg1, kr1, bh2, wht, be)
    s2p = _sc_scatter_add(k2, dst3, n_pad, nc, ns)
    s2 = _tc_merge(s2p, bn_pad)
    sg2, kr2 = _sc_gather2(s2, src3, k2, rev3, e, nc)
    h3 = _tc_update(h0, sg2, kr2, bh2, be)
    sfp = _sc_scatter_add(h3, dst3, n_pad, nc, ns)
    return _tc_final(x, sfp, woxt, womt, bo2, bn)


# speedup vs baseline: 2.2824x; 1.1460x over previous
import functools

import jax
import jax.numpy as jnp
from jax import lax
from jax.experimental import pallas as pl
from jax.experimental.pallas import tpu as pltpu
from jax.experimental.pallas import tpu_sc as plsc

F32 = jnp.float32


def _mm_body(x_ref, w_ref, o_ref):
    o_ref[...] = jnp.dot(x_ref[...], w_ref[...], preferred_element_type=F32)


def _tc_matmul(x, w, bn):
    n, d = x.shape
    h = w.shape[1]
    return pl.pallas_call(
        _mm_body,
        grid=(n // bn,),
        in_specs=[
            pl.BlockSpec((bn, d), lambda i: (i, 0)),
            pl.BlockSpec((d, h), lambda i: (0, 0)),
        ],
        out_specs=pl.BlockSpec((bn, h), lambda i: (i, 0)),
        out_shape=jax.ShapeDtypeStruct((n, h), F32),
    )(x, w)


def _h0k1_body(ea_ref, pg_ref, wet_ref, bi_ref, wht_ref, h0_ref, k1_ref):
    h0 = (
        jnp.dot(ea_ref[...], wet_ref[...], preferred_element_type=F32)
        + bi_ref[...]
        + pg_ref[...]
    )
    h0_ref[...] = h0
    k1_ref[...] = jnp.dot(
        jnp.maximum(h0, 0.0), wht_ref[...], preferred_element_type=F32
    )


def _tc_h0k1(ea, pg, wet, bi2, wht, be):
    e, de = ea.shape
    hid = wet.shape[1]
    out_sd = jax.ShapeDtypeStruct((e, hid), F32)
    return pl.pallas_call(
        _h0k1_body,
        grid=(e // be,),
        in_specs=[
            pl.BlockSpec((be, de), lambda i: (i, 0)),
            pl.BlockSpec((be, hid), lambda i: (i, 0)),
            pl.BlockSpec((de, hid), lambda i: (0, 0)),
            pl.BlockSpec((1, hid), lambda i: (0, 0)),
            pl.BlockSpec((hid, hid), lambda i: (0, 0)),
        ],
        out_specs=[
            pl.BlockSpec((be, hid), lambda i: (i, 0)),
            pl.BlockSpec((be, hid), lambda i: (i, 0)),
        ],
        out_shape=[out_sd, out_sd],
    )(ea, pg, wet, bi2, wht)


def _upd_mm_body(h0_ref, sg_ref, kr_ref, bh_ref, wht_ref, k_ref):
    h = jnp.maximum(h0_ref[...] + sg_ref[...] - kr_ref[...] + bh_ref[...], 0.0)
    k_ref[...] = jnp.dot(h, wht_ref[...], preferred_element_type=F32)


def _tc_update_matmul(h0, sg, kr, bh2, wht, be):
    e, hid = h0.shape
    return pl.pallas_call(
        _upd_mm_body,
        grid=(e // be,),
        in_specs=[
            pl.BlockSpec((be, hid), lambda i: (i, 0)),
            pl.BlockSpec((be, hid), lambda i: (i, 0)),
            pl.BlockSpec((be, hid), lambda i: (i, 0)),
            pl.BlockSpec((1, hid), lambda i: (0, 0)),
            pl.BlockSpec((hid, hid), lambda i: (0, 0)),
        ],
        out_specs=pl.BlockSpec((be, hid), lambda i: (i, 0)),
        out_shape=jax.ShapeDtypeStruct((e, hid), F32),
    )(h0, sg, kr, bh2, wht)


def _upd_body(h0_ref, sg_ref, kr_ref, bh_ref, h_ref):
    h_ref[...] = jnp.maximum(h0_ref[...] + sg_ref[...] - kr_ref[...] + bh_ref[...], 0.0)


def _tc_update(h0, sg, kr, bh2, be):
    e, hid = h0.shape
    return pl.pallas_call(
        _upd_body,
        grid=(e // be,),
        in_specs=[
            pl.BlockSpec((be, hid), lambda i: (i, 0)),
            pl.BlockSpec((be, hid), lambda i: (i, 0)),
            pl.BlockSpec((be, hid), lambda i: (i, 0)),
            pl.BlockSpec((1, hid), lambda i: (0, 0)),
        ],
        out_specs=pl.BlockSpec((be, hid), lambda i: (i, 0)),
        out_shape=jax.ShapeDtypeStruct((e, hid), F32),
    )(h0, sg, kr, bh2)


def _merge_body(p_ref, o_ref):
    o_ref[...] = p_ref[0] + p_ref[1]


def _tc_merge(parts, bn):
    nc, n, hid = parts.shape
    return pl.pallas_call(
        _merge_body,
        grid=(n // bn,),
        in_specs=[pl.BlockSpec((nc, bn, hid), lambda i: (0, i, 0))],
        out_specs=pl.BlockSpec((bn, hid), lambda i: (i, 0)),
        out_shape=jax.ShapeDtypeStruct((n, hid), F32),
    )(parts)


def _final_body(x_ref, sfp_ref, woxt_ref, womt_ref, bo_ref, o_ref):
    sf = sfp_ref[0] + sfp_ref[1]
    rs = jnp.sum(sf, axis=1, keepdims=True)
    m = jnp.where(rs == 0.0, x_ref[...], sf)
    o_ref[...] = jnp.maximum(
        jnp.dot(x_ref[...], woxt_ref[...], preferred_element_type=F32)
        + jnp.dot(m, womt_ref[...], preferred_element_type=F32)
        + bo_ref[...],
        0.0,
    )


def _tc_final(x, sfp, woxt, womt, bo2, bn):
    n, d = x.shape
    hid = womt.shape[1]
    return pl.pallas_call(
        _final_body,
        grid=(n // bn,),
        in_specs=[
            pl.BlockSpec((bn, d), lambda i: (i, 0)),
            pl.BlockSpec((2, bn, hid), lambda i: (0, i, 0)),
            pl.BlockSpec((d, hid), lambda i: (0, 0)),
            pl.BlockSpec((hid, hid), lambda i: (0, 0)),
            pl.BlockSpec((1, hid), lambda i: (0, 0)),
        ],
        out_specs=pl.BlockSpec((bn, hid), lambda i: (i, 0)),
        out_shape=jax.ShapeDtypeStruct((n, hid), F32),
    )(x, sfp, woxt, womt, bo2)


def _sc_gather1(table, idx3, e_total, nc):
    nw, chunks, c = idx3.shape
    perw = chunks * c
    hid = table.shape[1]
    mesh = plsc.VectorSubcoreMesh(core_axis_name="c", subcore_axis_name="s")

    half = (chunks + 1) // 2

    @functools.partial(
        pl.kernel,
        mesh=mesh,
        out_type=jax.ShapeDtypeStruct((e_total, hid), F32),
        scratch_types=[
            pltpu.VMEM((chunks, c), jnp.int32),
            pltpu.VMEM((2, c, hid), F32),
            pltpu.SemaphoreType.DMA,
            pltpu.SemaphoreType.DMA,
            pltpu.SemaphoreType.DMA,
        ],
    )
    def k(table_h, idx_h, out_h, idx_v, rows_v, sem_g, sem_w0, sem_w1):
        cid = lax.axis_index("c")
        sid = lax.axis_index("s")
        wid = sid * nc + cid
        sem_w = (sem_w0, sem_w1)
        pltpu.sync_copy(idx_h.at[wid], idx_v)

        def body(g, carry):
            for s in range(2):
                ci = g * 2 + s

                def step(ci=ci, s=s):
                    base = wid * perw + ci * c

                    @pl.when(g >= 1)
                    def _():
                        pltpu.make_async_copy(
                            rows_v.at[s], out_h.at[pl.ds(base, c)], sem_w[s]
                        ).wait()

                    pltpu.async_copy(table_h.at[idx_v.at[ci]], rows_v.at[s], sem_g).wait()
                    pltpu.async_copy(rows_v.at[s], out_h.at[pl.ds(base, c)], sem_w[s])

                if s == 0:
                    step()
                else:
                    pl.when(ci < chunks)(step)
            return carry

        lax.fori_loop(0, half, body, 0)
        for s in range(2):
            pltpu.make_async_copy(rows_v.at[s], out_h.at[pl.ds(0, c)], sem_w[s]).wait()

    return k(table, idx3)


def _sc_gather2(table_a, idx_a3, table_b, idx_b3, e_total, nc):
    nw, chunks, c = idx_a3.shape
    perw = chunks * c
    hid = table_a.shape[1]
    mesh = plsc.VectorSubcoreMesh(core_axis_name="c", subcore_axis_name="s")
    out_sd = jax.ShapeDtypeStruct((e_total, hid), F32)

    half = (chunks + 1) // 2

    @functools.partial(
        pl.kernel,
        mesh=mesh,
        out_type=[out_sd, out_sd],
        scratch_types=[
            pltpu.VMEM((chunks, c), jnp.int32),
            pltpu.VMEM((chunks, c), jnp.int32),
            pltpu.VMEM((2, c, hid), F32),
            pltpu.VMEM((2, c, hid), F32),
            pltpu.SemaphoreType.DMA,
            pltpu.SemaphoreType.DMA,
            pltpu.SemaphoreType.DMA,
            pltpu.SemaphoreType.DMA,
        ],
    )
    def k(ta_h, ia_h, tb_h, ib_h, oa_h, ob_h, ia_v, ib_v, a_v, b_v,
          sem_a, sem_b, sem_w0, sem_w1):
        cid = lax.axis_index("c")
        sid = lax.axis_index("s")
        wid = sid * nc + cid
        sem_w = (sem_w0, sem_w1)
        pltpu.sync_copy(ia_h.at[wid], ia_v)
        pltpu.sync_copy(ib_h.at[wid], ib_v)

        def body(g, carry):
            for s in range(2):
                ci = g * 2 + s

                def step(ci=ci, s=s):
                    base = wid * perw + ci * c

                    @pl.when(g >= 1)
                    def _():
                        pltpu.make_async_copy(
                            a_v.at[s], oa_h.at[pl.ds(base, c)], sem_w[s]
                        ).wait()
                        pltpu.make_async_copy(
                            b_v.at[s], ob_h.at[pl.ds(base, c)], sem_w[s]
                        ).wait()

                    cp_a = pltpu.async_copy(ta_h.at[ia_v.at[ci]], a_v.at[s], sem_a)
                    cp_b = pltpu.async_copy(tb_h.at[ib_v.at[ci]], b_v.at[s], sem_b)
                    cp_a.wait()
                    cp_b.wait()
                    pltpu.async_copy(a_v.at[s], oa_h.at[pl.ds(base, c)], sem_w[s])
                    pltpu.async_copy(b_v.at[s], ob_h.at[pl.ds(base, c)], sem_w[s])

                if s == 0:
                    step()
                else:
                    pl.when(ci < chunks)(step)
            return carry

        lax.fori_loop(0, half, body, 0)
        for s in range(2):
            pltpu.make_async_copy(a_v.at[s], oa_h.at[pl.ds(0, c)], sem_w[s]).wait()
            pltpu.make_async_copy(b_v.at[s], ob_h.at[pl.ds(0, c)], sem_w[s]).wait()

    return k(table_a, idx_a3, table_b, idx_b3)


def _sc_scatter_add(rows, dst3, n_pad, nc, ns):
    e_total, hid = rows.shape
    nw, chunks, c = dst3.shape
    perw = chunks * c
    stripe = n_pad // ns
    mesh = plsc.VectorSubcoreMesh(core_axis_name="c", subcore_axis_name="s")

    n_full, rem = stripe // c, stripe % c

    half = (chunks + 1) // 2

    @functools.partial(
        pl.kernel,
        mesh=mesh,
        out_type=jax.ShapeDtypeStruct((nc, n_pad, hid), F32),
        scratch_types=[
            pltpu.VMEM((chunks, c), jnp.int32),
            pltpu.VMEM((2, c, hid), F32),
            pltpu.VMEM_SHARED((n_pad, hid), F32),
            pltpu.SemaphoreType.DMA,
        ],
    )
    def k(rows_h, dst_h, out_h, idx_v, rows_v, acc_sh, sem):
        cid = lax.axis_index("c")
        sid = lax.axis_index("s")
        wid = sid * nc + cid
        z16 = jnp.zeros((16,), F32)
        zb = rows_v.at[0]

        def zrow(i, carry):
            for j in range(hid // 16):
                rows_v[0, i, pl.ds(j * 16, 16)] = z16
            return carry

        lax.fori_loop(0, c, zrow, 0)
        for t in range(n_full):
            pltpu.sync_copy(zb, acc_sh.at[pl.ds(sid * stripe + t * c, c)])
        if rem:
            pltpu.sync_copy(
                rows_v.at[0, pl.ds(0, rem)],
                acc_sh.at[pl.ds(sid * stripe + n_full * c, rem)],
            )
        plsc.subcore_barrier()

        pltpu.sync_copy(dst_h.at[wid], idx_v)
        pltpu.async_copy(rows_h.at[pl.ds(wid * perw, c)], rows_v.at[0], sem)

        def body(g, carry):
            for s in range(2):
                ci = g * 2 + s

                def step(ci=ci, s=s):
                    pltpu.make_async_copy(
                        rows_h.at[pl.ds(wid * perw, c)], rows_v.at[s], sem
                    ).wait()

                    @pl.when(ci + 1 < chunks)
                    def _():
                        base_n = wid * perw + (ci + 1) * c
                        pltpu.async_copy(
                            rows_h.at[pl.ds(base_n, c)], rows_v.at[1 - s], sem
                        )

                    pltpu.sync_copy(rows_v.at[s], acc_sh.at[idx_v.at[ci]], add=True)

                if s == 0:
                    step()
                else:
                    pl.when(ci < chunks)(step)
            return carry

        lax.fori_loop(0, half, body, 0)
        plsc.subcore_barrier()

        for t in range(n_full):
            pltpu.sync_copy(acc_sh.at[pl.ds(sid * stripe + t * c, c)], zb)
            pltpu.sync_copy(zb, out_h.at[cid, pl.ds(sid * stripe + t * c, c)])
        if rem:
            pltpu.sync_copy(
                acc_sh.at[pl.ds(sid * stripe + n_full * c, rem)],
                rows_v.at[0, pl.ds(0, rem)],
            )
            pltpu.sync_copy(
                rows_v.at[0, pl.ds(0, rem)],
                out_h.at[cid, pl.ds(sid * stripe + n_full * c, rem)],
            )

    return k(rows, dst3)


def kernel(x, edge_attr, W_i_w, W_i_b, W_h_w, W_h_b, W_o_w, W_o_b, edge_index, rev_edge_index):
    n, d = x.shape
    e, de = edge_attr.shape
    hid = W_i_w.shape[0]

    info = plsc.get_sparse_core_info()
    nc, ns = info.num_cores, info.num_subcores
    nw = nc * ns
    perw = e // nw
    c = 80
    chunks = perw // c

    src = edge_index[0].astype(jnp.int32)
    dst = edge_index[1].astype(jnp.int32)
    rev = rev_edge_index.astype(jnp.int32)
    src3 = src.reshape(nw, chunks, c)
    dst3 = dst.reshape(nw, chunks, c)
    rev3 = rev.reshape(nw, chunks, c)

    wxt = W_i_w[:, :d].T
    wet = W_i_w[:, d:].T
    wht = W_h_w.T
    woxt = W_o_w[:, :d].T
    womt = W_o_w[:, d:].T
    bi2 = W_i_b.reshape(1, hid)
    bh2 = W_h_b.reshape(1, hid)
    bo2 = W_o_b.reshape(1, hid)

    bn = 1000
    be = 1000
    stripe = (-(-n // ns) + 7) // 8 * 8
    n_pad = stripe * ns
    bn_pad = 2 * stripe

    p = _tc_matmul(x, wxt, bn)
    pg = _sc_gather1(p, src3, e, nc)
    h0, k1 = _tc_h0k1(edge_attr, pg, wet, bi2, wht, be)
    s1p = _sc_scatter_add(k1, dst3, n_pad, nc, ns)
    s1 = _tc_merge(s1p, bn_pad)
    sg1, kr1 = _sc_gather2(s1, src3, k1, rev3, e, nc)
    k2 = _tc_update_matmul(h0, sg1, kr1, bh2, wht, be)
    s2p = _sc_scatter_add(k2, dst3, n_pad, nc, ns)
    s2 = _tc_merge(s2p, bn_pad)
    sg2, kr2 = _sc_gather2(s2, src3, k2, rev3, e, nc)
    h3 = _tc_update(h0, sg2, kr2, bh2, be)
    sfp = _sc_scatter_add(h3, dst3, n_pad, nc, ns)
    return _tc_final(x, sfp, woxt, womt, bo2, bn)


# speedup vs baseline: 2.3903x; 1.0473x over previous
import functools

import jax
import jax.numpy as jnp
from jax import lax
from jax.experimental import pallas as pl
from jax.experimental.pallas import tpu as pltpu
from jax.experimental.pallas import tpu_sc as plsc

F32 = jnp.float32


def _mm_body(x_ref, w_ref, o_ref):
    o_ref[...] = jnp.dot(x_ref[...], w_ref[...], preferred_element_type=F32)


def _tc_matmul(x, w, bn):
    n, d = x.shape
    h = w.shape[1]
    return pl.pallas_call(
        _mm_body,
        grid=(n // bn,),
        in_specs=[
            pl.BlockSpec((bn, d), lambda i: (i, 0)),
            pl.BlockSpec((d, h), lambda i: (0, 0)),
        ],
        out_specs=pl.BlockSpec((bn, h), lambda i: (i, 0)),
        out_shape=jax.ShapeDtypeStruct((n, h), F32),
    )(x, w)


def _k1_body(ea_ref, pg_ref, wet_ref, bi_ref, wht_ref, k1_ref):
    h0 = (
        jnp.dot(ea_ref[...], wet_ref[...], preferred_element_type=F32)
        + bi_ref[...]
        + pg_ref[...]
    )
    k1_ref[...] = jnp.dot(
        jnp.maximum(h0, 0.0), wht_ref[...], preferred_element_type=F32
    )


def _tc_k1(ea, pg, wet, bi2, wht, be):
    e, de = ea.shape
    hid = wet.shape[1]
    return pl.pallas_call(
        _k1_body,
        grid=(e // be,),
        in_specs=[
            pl.BlockSpec((be, de), lambda i: (i, 0)),
            pl.BlockSpec((be, hid), lambda i: (i, 0)),
            pl.BlockSpec((de, hid), lambda i: (0, 0)),
            pl.BlockSpec((1, hid), lambda i: (0, 0)),
            pl.BlockSpec((hid, hid), lambda i: (0, 0)),
        ],
        out_specs=pl.BlockSpec((be, hid), lambda i: (i, 0)),
        out_shape=jax.ShapeDtypeStruct((e, hid), F32),
    )(ea, pg, wet, bi2, wht)


def _upd_mm_body(ea_ref, pg_ref, sg_ref, kr_ref, wet_ref, bi_ref, bh_ref, wht_ref, k_ref):
    h0 = (
        jnp.dot(ea_ref[...], wet_ref[...], preferred_element_type=F32)
        + bi_ref[...]
        + pg_ref[...]
    )
    m = sg_ref[...] - kr_ref[...]
    h = jnp.maximum(h0 + m + bh_ref[...], 0.0)
    k_ref[...] = jnp.dot(h, wht_ref[...], preferred_element_type=F32)


def _tc_update_matmul(ea, pg, sg, kr, wet, bi2, bh2, wht, be):
    e, de = ea.shape
    hid = wet.shape[1]
    return pl.pallas_call(
        _upd_mm_body,
        grid=(e // be,),
        in_specs=[
            pl.BlockSpec((be, de), lambda i: (i, 0)),
            pl.BlockSpec((be, hid), lambda i: (i, 0)),
            pl.BlockSpec((be, hid), lambda i: (i, 0)),
            pl.BlockSpec((be, hid), lambda i: (i, 0)),
            pl.BlockSpec((de, hid), lambda i: (0, 0)),
            pl.BlockSpec((1, hid), lambda i: (0, 0)),
            pl.BlockSpec((1, hid), lambda i: (0, 0)),
            pl.BlockSpec((hid, hid), lambda i: (0, 0)),
        ],
        out_specs=pl.BlockSpec((be, hid), lambda i: (i, 0)),
        out_shape=jax.ShapeDtypeStruct((e, hid), F32),
    )(ea, pg, sg, kr, wet, bi2, bh2, wht)


def _upd_body(ea_ref, pg_ref, sg_ref, kr_ref, wet_ref, bi_ref, bh_ref, h_ref):
    h0 = (
        jnp.dot(ea_ref[...], wet_ref[...], preferred_element_type=F32)
        + bi_ref[...]
        + pg_ref[...]
    )
    m = sg_ref[...] - kr_ref[...]
    h_ref[...] = jnp.maximum(h0 + m + bh_ref[...], 0.0)


def _tc_update(ea, pg, sg, kr, wet, bi2, bh2, be):
    e, de = ea.shape
    hid = wet.shape[1]
    return pl.pallas_call(
        _upd_body,
        grid=(e // be,),
        in_specs=[
            pl.BlockSpec((be, de), lambda i: (i, 0)),
            pl.BlockSpec((be, hid), lambda i: (i, 0)),
            pl.BlockSpec((be, hid), lambda i: (i, 0)),
            pl.BlockSpec((be, hid), lambda i: (i, 0)),
            pl.BlockSpec((de, hid), lambda i: (0, 0)),
            pl.BlockSpec((1, hid), lambda i: (0, 0)),
            pl.BlockSpec((1, hid), lambda i: (0, 0)),
        ],
        out_specs=pl.BlockSpec((be, hid), lambda i: (i, 0)),
        out_shape=jax.ShapeDtypeStruct((e, hid), F32),
    )(ea, pg, sg, kr, wet, bi2, bh2)


def _merge_body(p_ref, o_ref):
    o_ref[...] = p_ref[0] + p_ref[1]


def _tc_merge(parts, bn):
    nc, n, hid = parts.shape
    return pl.pallas_call(
        _merge_body,
        grid=(n // bn,),
        in_specs=[pl.BlockSpec((nc, bn, hid), lambda i: (0, i, 0))],
        out_specs=pl.BlockSpec((bn, hid), lambda i: (i, 0)),
        out_shape=jax.ShapeDtypeStruct((n, hid), F32),
    )(parts)


def _final_body(x_ref, sfp_ref, woxt_ref, womt_ref, bo_ref, o_ref):
    sf = sfp_ref[0] + sfp_ref[1]
    rs = jnp.sum(sf, axis=1, keepdims=True)
    m = jnp.where(rs == 0.0, x_ref[...], sf)
    o_ref[...] = jnp.maximum(
        jnp.dot(x_ref[...], woxt_ref[...], preferred_element_type=F32)
        + jnp.dot(m, womt_ref[...], preferred_element_type=F32)
        + bo_ref[...],
        0.0,
    )


def _tc_final(x, sfp, woxt, womt, bo2, bn):
    n, d = x.shape
    hid = womt.shape[1]
    return pl.pallas_call(
        _final_body,
        grid=(n // bn,),
        in_specs=[
            pl.BlockSpec((bn, d), lambda i: (i, 0)),
            pl.BlockSpec((2, bn, hid), lambda i: (0, i, 0)),
            pl.BlockSpec((d, hid), lambda i: (0, 0)),
            pl.BlockSpec((hid, hid), lambda i: (0, 0)),
            pl.BlockSpec((1, hid), lambda i: (0, 0)),
        ],
        out_specs=pl.BlockSpec((bn, hid), lambda i: (i, 0)),
        out_shape=jax.ShapeDtypeStruct((n, hid), F32),
    )(x, sfp, woxt, womt, bo2)


def _sc_gather1(table, idx3, e_total, nc):
    nw, chunks, c = idx3.shape
    perw = chunks * c
    hid = table.shape[1]
    dt = table.dtype
    mesh = plsc.VectorSubcoreMesh(core_axis_name="c", subcore_axis_name="s")

    half = (chunks + 1) // 2

    @functools.partial(
        pl.kernel,
        mesh=mesh,
        out_type=jax.ShapeDtypeStruct((e_total, hid), dt),
        scratch_types=[
            pltpu.VMEM((chunks, c), jnp.int32),
            pltpu.VMEM((2, c, hid), dt),
            pltpu.SemaphoreType.DMA,
            pltpu.SemaphoreType.DMA,
            pltpu.SemaphoreType.DMA,
        ],
    )
    def k(table_h, idx_h, out_h, idx_v, rows_v, sem_g, sem_w0, sem_w1):
        cid = lax.axis_index("c")
        sid = lax.axis_index("s")
        wid = sid * nc + cid
        sem_w = (sem_w0, sem_w1)
        pltpu.sync_copy(idx_h.at[wid], idx_v)

        def body(g, carry):
            for s in range(2):
                ci = g * 2 + s

                def step(ci=ci, s=s):
                    base = wid * perw + ci * c

                    @pl.when(g >= 1)
                    def _():
                        pltpu.make_async_copy(
                            rows_v.at[s], out_h.at[pl.ds(base, c)], sem_w[s]
                        ).wait()

                    pltpu.async_copy(table_h.at[idx_v.at[ci]], rows_v.at[s], sem_g).wait()
                    pltpu.async_copy(rows_v.at[s], out_h.at[pl.ds(base, c)], sem_w[s])

                if s == 0:
                    step()
                else:
                    pl.when(ci < chunks)(step)
            return carry

        lax.fori_loop(0, half, body, 0)
        for s in range(2):
            pltpu.make_async_copy(rows_v.at[s], out_h.at[pl.ds(0, c)], sem_w[s]).wait()

    return k(table, idx3)


def _sc_gather2(table_a, idx_a3, table_b, idx_b3, e_total, nc):
    nw, chunks, c = idx_a3.shape
    perw = chunks * c
    hid = table_a.shape[1]
    mesh = plsc.VectorSubcoreMesh(core_axis_name="c", subcore_axis_name="s")
    dt = table_a.dtype
    out_sd = jax.ShapeDtypeStruct((e_total, hid), dt)

    half = (chunks + 1) // 2

    @functools.partial(
        pl.kernel,
        mesh=mesh,
        out_type=[out_sd, out_sd],
        scratch_types=[
            pltpu.VMEM((chunks, c), jnp.int32),
            pltpu.VMEM((chunks, c), jnp.int32),
            pltpu.VMEM((2, c, hid), dt),
            pltpu.VMEM((2, c, hid), dt),
            pltpu.SemaphoreType.DMA,
            pltpu.SemaphoreType.DMA,
            pltpu.SemaphoreType.DMA,
            pltpu.SemaphoreType.DMA,
        ],
    )
    def k(ta_h, ia_h, tb_h, ib_h, oa_h, ob_h, ia_v, ib_v, a_v, b_v,
          sem_a, sem_b, sem_w0, sem_w1):
        cid = lax.axis_index("c")
        sid = lax.axis_index("s")
        wid = sid * nc + cid
        sem_w = (sem_w0, sem_w1)
        pltpu.sync_copy(ia_h.at[wid], ia_v)
        pltpu.sync_copy(ib_h.at[wid], ib_v)

        def body(g, carry):
            for s in range(2):
                ci = g * 2 + s

                def step(ci=ci, s=s):
                    base = wid * perw + ci * c

                    @pl.when(g >= 1)
                    def _():
                        pltpu.make_async_copy(
                            a_v.at[s], oa_h.at[pl.ds(base, c)], sem_w[s]
                        ).wait()
                        pltpu.make_async_copy(
                            b_v.at[s], ob_h.at[pl.ds(base, c)], sem_w[s]
                        ).wait()

                    cp_a = pltpu.async_copy(ta_h.at[ia_v.at[ci]], a_v.at[s], sem_a)
                    cp_b = pltpu.async_copy(tb_h.at[ib_v.at[ci]], b_v.at[s], sem_b)
                    cp_a.wait()
                    cp_b.wait()
                    pltpu.async_copy(a_v.at[s], oa_h.at[pl.ds(base, c)], sem_w[s])
                    pltpu.async_copy(b_v.at[s], ob_h.at[pl.ds(base, c)], sem_w[s])

                if s == 0:
                    step()
                else:
                    pl.when(ci < chunks)(step)
            return carry

        lax.fori_loop(0, half, body, 0)
        for s in range(2):
            pltpu.make_async_copy(a_v.at[s], oa_h.at[pl.ds(0, c)], sem_w[s]).wait()
            pltpu.make_async_copy(b_v.at[s], ob_h.at[pl.ds(0, c)], sem_w[s]).wait()

    return k(table_a, idx_a3, table_b, idx_b3)


def _sc_scatter_add(rows, dst3, n_pad, nc, ns):
    e_total, hid = rows.shape
    nw, chunks, c = dst3.shape
    perw = chunks * c
    stripe = n_pad // ns
    mesh = plsc.VectorSubcoreMesh(core_axis_name="c", subcore_axis_name="s")

    n_full, rem = stripe // c, stripe % c

    half = (chunks + 1) // 2

    @functools.partial(
        pl.kernel,
        mesh=mesh,
        out_type=jax.ShapeDtypeStruct((nc, n_pad, hid), F32),
        scratch_types=[
            pltpu.VMEM((chunks, c), jnp.int32),
            pltpu.VMEM((2, c, hid), F32),
            pltpu.VMEM_SHARED((n_pad, hid), F32),
            pltpu.SemaphoreType.DMA,
        ],
    )
    def k(rows_h, dst_h, out_h, idx_v, rows_v, acc_sh, sem):
        cid = lax.axis_index("c")
        sid = lax.axis_index("s")
        wid = sid * nc + cid
        z16 = jnp.zeros((16,), F32)
        zb = rows_v.at[0]

        def zrow(i, carry):
            for j in range(hid // 16):
                rows_v[0, i, pl.ds(j * 16, 16)] = z16
            return carry

        lax.fori_loop(0, c, zrow, 0)
        for t in range(n_full):
            pltpu.sync_copy(zb, acc_sh.at[pl.ds(sid * stripe + t * c, c)])
        if rem:
            pltpu.sync_copy(
                rows_v.at[0, pl.ds(0, rem)],
                acc_sh.at[pl.ds(sid * stripe + n_full * c, rem)],
            )
        plsc.subcore_barrier()

        pltpu.sync_copy(dst_h.at[wid], idx_v)
        pltpu.async_copy(rows_h.at[pl.ds(wid * perw, c)], rows_v.at[0], sem)

        def body(g, carry):
            for s in range(2):
                ci = g * 2 + s

                def step(ci=ci, s=s):
                    pltpu.make_async_copy(
                        rows_h.at[pl.ds(wid * perw, c)], rows_v.at[s], sem
                    ).wait()

                    @pl.when(ci + 1 < chunks)
                    def _():
                        base_n = wid * perw + (ci + 1) * c
                        pltpu.async_copy(
                            rows_h.at[pl.ds(base_n, c)], rows_v.at[1 - s], sem
                        )

                    pltpu.sync_copy(rows_v.at[s], acc_sh.at[idx_v.at[ci]], add=True)

                if s == 0:
                    step()
                else:
                    pl.when(ci < chunks)(step)
            return carry

        lax.fori_loop(0, half, body, 0)
        plsc.subcore_barrier()

        for t in range(n_full):
            pltpu.sync_copy(acc_sh.at[pl.ds(sid * stripe + t * c, c)], zb)
            pltpu.sync_copy(zb, out_h.at[cid, pl.ds(sid * stripe + t * c, c)])
        if rem:
            pltpu.sync_copy(
                acc_sh.at[pl.ds(sid * stripe + n_full * c, rem)],
                rows_v.at[0, pl.ds(0, rem)],
            )
            pltpu.sync_copy(
                rows_v.at[0, pl.ds(0, rem)],
                out_h.at[cid, pl.ds(sid * stripe + n_full * c, rem)],
            )

    return k(rows, dst3)


def kernel(x, edge_attr, W_i_w, W_i_b, W_h_w, W_h_b, W_o_w, W_o_b, edge_index, rev_edge_index):
    n, d = x.shape
    e, de = edge_attr.shape
    hid = W_i_w.shape[0]

    info = plsc.get_sparse_core_info()
    nc, ns = info.num_cores, info.num_subcores
    nw = nc * ns
    perw = e // nw
    c = 80
    chunks = perw // c

    src = edge_index[0].astype(jnp.int32)
    dst = edge_index[1].astype(jnp.int32)
    rev = rev_edge_index.astype(jnp.int32)
    src3 = src.reshape(nw, chunks, c)
    dst3 = dst.reshape(nw, chunks, c)
    rev3 = rev.reshape(nw, chunks, c)

    wxt = W_i_w[:, :d].T
    wet = W_i_w[:, d:].T
    wht = W_h_w.T
    woxt = W_o_w[:, :d].T
    womt = W_o_w[:, d:].T
    bi2 = W_i_b.reshape(1, hid)
    bh2 = W_h_b.reshape(1, hid)
    bo2 = W_o_b.reshape(1, hid)

    bn = 1000
    be = 1280
    stripe = (-(-n // ns) + 7) // 8 * 8
    n_pad = stripe * ns
    bn_pad = 2 * stripe

    p = _tc_matmul(x, wxt, bn)
    pg = _sc_gather1(p, src3, e, nc)
    k1 = _tc_k1(edge_attr, pg, wet, bi2, wht, be)
    s1p = _sc_scatter_add(k1, dst3, n_pad, nc, ns)
    s1 = _tc_merge(s1p, bn_pad)
    sg1, kr1 = _sc_gather2(s1, src3, k1, rev3, e, nc)
    k2 = _tc_update_matmul(edge_attr, pg, sg1, kr1, wet, bi2, bh2, wht, be)
    s2p = _sc_scatter_add(k2, dst3, n_pad, nc, ns)
    s2 = _tc_merge(s2p, bn_pad)
    sg2, kr2 = _sc_gather2(s2, src3, k2, rev3, e, nc)
    h3 = _tc_update(edge_attr, pg, sg2, kr2, wet, bi2, bh2, be)
    sfp = _sc_scatter_add(h3, dst3, n_pad, nc, ns)
    return _tc_final(x, sfp, woxt, womt, bo2, bn)
